# baseline (speedup 1.0000x reference)
import functools

import jax
import jax.numpy as jnp
from jax import lax
from jax.experimental import pallas as pl

B = 2048
PROX = 1024
H = 256
DIST = 256
DEN = 16
K = 32

BB = 128
H_CHUNK = 8
CH = H_CHUNK * DIST
NEG = -3.4e38
POS = 3.4e38


def _tc_body(x_ref, d_ref, w_ref, b_ref, a_ref, o_ref):
    prox = lax.dot_general(
        x_ref[...], w_ref[...], (((1,), (1,)), ((), ())),
        preferred_element_type=jnp.float32)
    prox = prox + b_ref[...]

    d = d_ref[...]

    def step(i, carry):
        mpos, mneg = carry
        a_chunk = a_ref[:, pl.ds(i * CH, CH)]
        v = lax.dot_general(
            d, a_chunk, (((1,), (0,)), ((), ())),
            preferred_element_type=jnp.float32)
        hi = v[:, 0:DIST]
        lo = v[:, 0:DIST]
        for j in range(1, H_CHUNK):
            s = v[:, j * DIST:(j + 1) * DIST]
            hi = jnp.maximum(hi, s)
            lo = jnp.minimum(lo, s)
        return jnp.maximum(mpos, hi), jnp.minimum(mneg, lo)

    mpos, mneg = lax.fori_loop(
        0, H // H_CHUNK, step,
        (jnp.full((BB, DIST), NEG, jnp.float32),
         jnp.full((BB, DIST), POS, jnp.float32)))

    v = jnp.where(mpos >= -mneg, mpos, mneg)
    mod = 1.0 / (1.0 + jnp.exp(-v))
    res = prox * mod

    def peel(j, cur):
        m = jnp.max(cur, axis=1, keepdims=True)
        return jnp.where(cur == m, NEG, cur)

    cur = lax.fori_loop(0, K - 1, peel, res)
    thr = jnp.max(cur, axis=1, keepdims=True)
    o_ref[...] = jnp.where(res >= thr, res, 0.0)


@jax.jit
def _run(x_bf, d_bf, W_bf, b2d, A2_bf):
    return pl.pallas_call(
        _tc_body,
        grid=(B // BB,),
        in_specs=[
            pl.BlockSpec((BB, PROX), lambda i: (i, 0)),
            pl.BlockSpec((BB, DEN), lambda i: (i, 0)),
            pl.BlockSpec((H, PROX), lambda i: (0, 0)),
            pl.BlockSpec((1, H), lambda i: (0, 0)),
            pl.BlockSpec((DEN, H * DIST), lambda i: (0, 0)),
        ],
        out_specs=pl.BlockSpec((BB, H), lambda i: (i, 0)),
        out_shape=jax.ShapeDtypeStruct((B, H), jnp.float32),
    )(x_bf, d_bf, W_bf, b2d, A2_bf)


def kernel(proximal_input, distal_input, W, b, distal):
    A2 = jnp.transpose(distal, (1, 0, 2)).reshape(DEN, H * DIST)
    return _run(proximal_input.astype(jnp.bfloat16),
                distal_input.astype(jnp.bfloat16),
                W.astype(jnp.bfloat16),
                b.reshape(1, H),
                A2.astype(jnp.bfloat16))


# speedup vs baseline: 1.4969x; 1.4969x over previous
import functools

import jax
import jax.numpy as jnp
from jax import lax
from jax.experimental import pallas as pl

B = 2048
PROX = 1024
H = 256
DIST = 256
DEN = 16
K = 32

BB = 256
H_CHUNK = 16
CH = H_CHUNK * DIST
NEG = -3.4e38
POS = 3.4e38


def _tc_body(x_ref, d_ref, w_ref, b_ref, a_ref, o_ref):
    prox = lax.dot_general(
        x_ref[...], w_ref[...], (((1,), (1,)), ((), ())),
        preferred_element_type=jnp.float32)
    prox = prox + b_ref[...]

    d = d_ref[...]

    def step(i, carry):
        mpos, mneg = carry
        a_chunk = a_ref[:, pl.ds(i * CH, CH)]
        v = lax.dot_general(
            d, a_chunk, (((1,), (0,)), ((), ())),
            preferred_element_type=jnp.float32)
        hi = v[:, 0:DIST]
        lo = v[:, 0:DIST]
        for j in range(1, H_CHUNK):
            s = v[:, j * DIST:(j + 1) * DIST]
            hi = jnp.maximum(hi, s)
            lo = jnp.minimum(lo, s)
        return jnp.maximum(mpos, hi), jnp.minimum(mneg, lo)

    mpos, mneg = lax.fori_loop(
        0, H // H_CHUNK, step,
        (jnp.full((BB, DIST), NEG, jnp.float32),
         jnp.full((BB, DIST), POS, jnp.float32)))

    v = jnp.where(mpos >= -mneg, mpos, mneg)
    mod = 1.0 / (1.0 + jnp.exp(-v))
    res = prox * mod

    def peel(j, cur):
        m = jnp.max(cur, axis=1, keepdims=True)
        return jnp.where(cur == m, NEG, cur)

    cur = lax.fori_loop(0, K - 1, peel, res)
    thr = jnp.max(cur, axis=1, keepdims=True)
    o_ref[...] = jnp.where(res >= thr, res, 0.0)


@jax.jit
def _run(x_bf, d_bf, W_bf, b2d, A2_bf):
    return pl.pallas_call(
        _tc_body,
        grid=(B // BB,),
        in_specs=[
            pl.BlockSpec((BB, PROX), lambda i: (i, 0)),
            pl.BlockSpec((BB, DEN), lambda i: (i, 0)),
            pl.BlockSpec((H, PROX), lambda i: (0, 0)),
            pl.BlockSpec((1, H), lambda i: (0, 0)),
            pl.BlockSpec((DEN, H * DIST), lambda i: (0, 0)),
        ],
        out_specs=pl.BlockSpec((BB, H), lambda i: (i, 0)),
        out_shape=jax.ShapeDtypeStruct((B, H), jnp.float32),
    )(x_bf, d_bf, W_bf, b2d, A2_bf)


def kernel(proximal_input, distal_input, W, b, distal):
    A2 = jnp.transpose(distal, (1, 0, 2)).reshape(DEN, H * DIST)
    return _run(proximal_input.astype(jnp.bfloat16),
                distal_input.astype(jnp.bfloat16),
                W.astype(jnp.bfloat16),
                b.reshape(1, H),
                A2.astype(jnp.bfloat16))


# speedup vs baseline: 1.6147x; 1.0787x over previous
import functools

import jax
import jax.numpy as jnp
from jax import lax
from jax.experimental import pallas as pl

B = 2048
PROX = 1024
H = 256
DIST = 256
DEN = 16
K = 32

BB = 256
H_CHUNK = 32
CH = H_CHUNK * DIST
NEG = -3.4e38
POS = 3.4e38


def _tc_body(x_ref, d_ref, w_ref, b_ref, a_ref, o_ref):
    prox = lax.dot_general(
        x_ref[...], w_ref[...], (((1,), (1,)), ((), ())),
        preferred_element_type=jnp.float32)
    prox = prox + b_ref[...]

    d = d_ref[...]

    def step(i, carry):
        mpos, mneg = carry
        a_chunk = a_ref[:, pl.ds(i * CH, CH)]
        v = lax.dot_general(
            d, a_chunk, (((1,), (0,)), ((), ())),
            preferred_element_type=jnp.float32)
        hi = v[:, 0:DIST]
        lo = v[:, 0:DIST]
        for j in range(1, H_CHUNK):
            s = v[:, j * DIST:(j + 1) * DIST]
            hi = jnp.maximum(hi, s)
            lo = jnp.minimum(lo, s)
        return jnp.maximum(mpos, hi), jnp.minimum(mneg, lo)

    mpos, mneg = lax.fori_loop(
        0, H // H_CHUNK, step,
        (jnp.full((BB, DIST), NEG, jnp.float32),
         jnp.full((BB, DIST), POS, jnp.float32)))

    v = jnp.where(mpos >= -mneg, mpos, mneg)
    mod = 1.0 / (1.0 + jnp.exp(-v))
    res = prox * mod

    def peel(j, cur):
        m = jnp.max(cur, axis=1, keepdims=True)
        return jnp.where(cur == m, NEG, cur)

    cur = lax.fori_loop(0, K - 1, peel, res)
    thr = jnp.max(cur, axis=1, keepdims=True)
    o_ref[...] = jnp.where(res >= thr, res, 0.0)


@jax.jit
def _run(x_bf, d_bf, W_bf, b2d, A2_bf):
    return pl.pallas_call(
        _tc_body,
        grid=(B // BB,),
        in_specs=[
            pl.BlockSpec((BB, PROX), lambda i: (i, 0)),
            pl.BlockSpec((BB, DEN), lambda i: (i, 0)),
            pl.BlockSpec((H, PROX), lambda i: (0, 0)),
            pl.BlockSpec((1, H), lambda i: (0, 0)),
            pl.BlockSpec((DEN, H * DIST), lambda i: (0, 0)),
        ],
        out_specs=pl.BlockSpec((BB, H), lambda i: (i, 0)),
        out_shape=jax.ShapeDtypeStruct((B, H), jnp.float32),
    )(x_bf, d_bf, W_bf, b2d, A2_bf)


def kernel(proximal_input, distal_input, W, b, distal):
    A2 = jnp.transpose(distal, (1, 0, 2)).reshape(DEN, H * DIST)
    return _run(proximal_input.astype(jnp.bfloat16),
                distal_input.astype(jnp.bfloat16),
                W.astype(jnp.bfloat16),
                b.reshape(1, H),
                A2.astype(jnp.bfloat16))


# speedup vs baseline: 1.6872x; 1.0449x over previous
import functools

import jax
import jax.numpy as jnp
from jax import lax
from jax.experimental import pallas as pl

B = 2048
PROX = 1024
H = 256
DIST = 256
DEN = 16
K = 32

BB = 256
H_CHUNK = 64
CH = H_CHUNK * DIST
NEG = -3.4e38
POS = 3.4e38


def _tc_body(x_ref, d_ref, w_ref, b_ref, a_ref, o_ref):
    prox = lax.dot_general(
        x_ref[...], w_ref[...], (((1,), (1,)), ((), ())),
        preferred_element_type=jnp.float32)
    prox = prox + b_ref[...]

    d = d_ref[...]

    def step(i, carry):
        mpos, mneg = carry
        a_chunk = a_ref[:, pl.ds(i * CH, CH)]
        v = lax.dot_general(
            d, a_chunk, (((1,), (0,)), ((), ())),
            preferred_element_type=jnp.float32)
        hi = v[:, 0:DIST]
        lo = v[:, 0:DIST]
        for j in range(1, H_CHUNK):
            s = v[:, j * DIST:(j + 1) * DIST]
            hi = jnp.maximum(hi, s)
            lo = jnp.minimum(lo, s)
        return jnp.maximum(mpos, hi), jnp.minimum(mneg, lo)

    mpos, mneg = lax.fori_loop(
        0, H // H_CHUNK, step,
        (jnp.full((BB, DIST), NEG, jnp.float32),
         jnp.full((BB, DIST), POS, jnp.float32)))

    v = jnp.where(mpos >= -mneg, mpos, mneg)
    mod = 1.0 / (1.0 + jnp.exp(-v))
    res = prox * mod

    def peel(j, cur):
        m = jnp.max(cur, axis=1, keepdims=True)
        return jnp.where(cur == m, NEG, cur)

    cur = lax.fori_loop(0, K - 1, peel, res)
    thr = jnp.max(cur, axis=1, keepdims=True)
    o_ref[...] = jnp.where(res >= thr, res, 0.0)


@jax.jit
def _run(x_bf, d_bf, W_bf, b2d, A2_bf):
    return pl.pallas_call(
        _tc_body,
        grid=(B // BB,),
        in_specs=[
            pl.BlockSpec((BB, PROX), lambda i: (i, 0)),
            pl.BlockSpec((BB, DEN), lambda i: (i, 0)),
            pl.BlockSpec((H, PROX), lambda i: (0, 0)),
            pl.BlockSpec((1, H), lambda i: (0, 0)),
            pl.BlockSpec((DEN, H * DIST), lambda i: (0, 0)),
        ],
        out_specs=pl.BlockSpec((BB, H), lambda i: (i, 0)),
        out_shape=jax.ShapeDtypeStruct((B, H), jnp.float32),
    )(x_bf, d_bf, W_bf, b2d, A2_bf)


def kernel(proximal_input, distal_input, W, b, distal):
    A2 = jnp.transpose(distal, (1, 0, 2)).reshape(DEN, H * DIST)
    return _run(proximal_input.astype(jnp.bfloat16),
                distal_input.astype(jnp.bfloat16),
                W.astype(jnp.bfloat16),
                b.reshape(1, H),
                A2.astype(jnp.bfloat16))


# speedup vs baseline: 1.7206x; 1.0198x over previous
import functools

import jax
import jax.numpy as jnp
from jax import lax
from jax.experimental import pallas as pl

B = 2048
PROX = 1024
H = 256
DIST = 256
DEN = 16
K = 32

BB = 256
H_CHUNK = 128
CH = H_CHUNK * DIST
NEG = -3.4e38
POS = 3.4e38


def _tc_body(x_ref, d_ref, w_ref, b_ref, a_ref, o_ref):
    prox = lax.dot_general(
        x_ref[...], w_ref[...], (((1,), (1,)), ((), ())),
        preferred_element_type=jnp.float32)
    prox = prox + b_ref[...]

    d = d_ref[...]

    def step(i, carry):
        mpos, mneg = carry
        a_chunk = a_ref[:, pl.ds(i * CH, CH)]
        v = lax.dot_general(
            d, a_chunk, (((1,), (0,)), ((), ())),
            preferred_element_type=jnp.float32)
        hi = v[:, 0:DIST]
        lo = v[:, 0:DIST]
        for j in range(1, H_CHUNK):
            s = v[:, j * DIST:(j + 1) * DIST]
            hi = jnp.maximum(hi, s)
            lo = jnp.minimum(lo, s)
        return jnp.maximum(mpos, hi), jnp.minimum(mneg, lo)

    mpos, mneg = lax.fori_loop(
        0, H // H_CHUNK, step,
        (jnp.full((BB, DIST), NEG, jnp.float32),
         jnp.full((BB, DIST), POS, jnp.float32)))

    v = jnp.where(mpos >= -mneg, mpos, mneg)
    mod = 1.0 / (1.0 + jnp.exp(-v))
    res = prox * mod

    def peel(j, cur):
        m = jnp.max(cur, axis=1, keepdims=True)
        return jnp.where(cur == m, NEG, cur)

    cur = lax.fori_loop(0, K - 1, peel, res)
    thr = jnp.max(cur, axis=1, keepdims=True)
    o_ref[...] = jnp.where(res >= thr, res, 0.0)


@jax.jit
def _run(x_bf, d_bf, W_bf, b2d, A2_bf):
    return pl.pallas_call(
        _tc_body,
        grid=(B // BB,),
        in_specs=[
            pl.BlockSpec((BB, PROX), lambda i: (i, 0)),
            pl.BlockSpec((BB, DEN), lambda i: (i, 0)),
            pl.BlockSpec((H, PROX), lambda i: (0, 0)),
            pl.BlockSpec((1, H), lambda i: (0, 0)),
            pl.BlockSpec((DEN, H * DIST), lambda i: (0, 0)),
        ],
        out_specs=pl.BlockSpec((BB, H), lambda i: (i, 0)),
        out_shape=jax.ShapeDtypeStruct((B, H), jnp.float32),
    )(x_bf, d_bf, W_bf, b2d, A2_bf)


def kernel(proximal_input, distal_input, W, b, distal):
    A2 = jnp.transpose(distal, (1, 0, 2)).reshape(DEN, H * DIST)
    return _run(proximal_input.astype(jnp.bfloat16),
                distal_input.astype(jnp.bfloat16),
                W.astype(jnp.bfloat16),
                b.reshape(1, H),
                A2.astype(jnp.bfloat16))


# speedup vs baseline: 1.8116x; 1.0529x over previous
import functools

import jax
import jax.numpy as jnp
from jax import lax
from jax.experimental import pallas as pl

B = 2048
PROX = 1024
H = 256
DIST = 256
DEN = 16
K = 32

BB = 512
H_CHUNK = 64
CH = H_CHUNK * DIST
NEG = -3.4e38
POS = 3.4e38


def _tc_body(x_ref, d_ref, w_ref, b_ref, a_ref, o_ref):
    prox = lax.dot_general(
        x_ref[...], w_ref[...], (((1,), (1,)), ((), ())),
        preferred_element_type=jnp.float32)
    prox = prox + b_ref[...]

    d = d_ref[...]

    def step(i, carry):
        mpos, mneg = carry
        a_chunk = a_ref[:, pl.ds(i * CH, CH)]
        v = lax.dot_general(
            d, a_chunk, (((1,), (0,)), ((), ())),
            preferred_element_type=jnp.float32)
        hi = v[:, 0:DIST]
        lo = v[:, 0:DIST]
        for j in range(1, H_CHUNK):
            s = v[:, j * DIST:(j + 1) * DIST]
            hi = jnp.maximum(hi, s)
            lo = jnp.minimum(lo, s)
        return jnp.maximum(mpos, hi), jnp.minimum(mneg, lo)

    mpos, mneg = lax.fori_loop(
        0, H // H_CHUNK, step,
        (jnp.full((BB, DIST), NEG, jnp.float32),
         jnp.full((BB, DIST), POS, jnp.float32)))

    v = jnp.where(mpos >= -mneg, mpos, mneg)
    mod = 1.0 / (1.0 + jnp.exp(-v))
    res = prox * mod

    def peel(j, cur):
        m = jnp.max(cur, axis=1, keepdims=True)
        return jnp.where(cur == m, NEG, cur)

    cur = lax.fori_loop(0, K - 1, peel, res)
    thr = jnp.max(cur, axis=1, keepdims=True)
    o_ref[...] = jnp.where(res >= thr, res, 0.0)


@jax.jit
def _run(x_bf, d_bf, W_bf, b2d, A2_bf):
    return pl.pallas_call(
        _tc_body,
        grid=(B // BB,),
        in_specs=[
            pl.BlockSpec((BB, PROX), lambda i: (i, 0)),
            pl.BlockSpec((BB, DEN), lambda i: (i, 0)),
            pl.BlockSpec((H, PROX), lambda i: (0, 0)),
            pl.BlockSpec((1, H), lambda i: (0, 0)),
            pl.BlockSpec((DEN, H * DIST), lambda i: (0, 0)),
        ],
        out_specs=pl.BlockSpec((BB, H), lambda i: (i, 0)),
        out_shape=jax.ShapeDtypeStruct((B, H), jnp.float32),
    )(x_bf, d_bf, W_bf, b2d, A2_bf)


def kernel(proximal_input, distal_input, W, b, distal):
    A2 = jnp.transpose(distal, (1, 0, 2)).reshape(DEN, H * DIST)
    return _run(proximal_input.astype(jnp.bfloat16),
                distal_input.astype(jnp.bfloat16),
                W.astype(jnp.bfloat16),
                b.reshape(1, H),
                A2.astype(jnp.bfloat16))


# speedup vs baseline: 1.8611x; 1.0273x over previous
import functools

import jax
import jax.numpy as jnp
from jax import lax
from jax.experimental import pallas as pl

B = 2048
PROX = 1024
H = 256
DIST = 256
DEN = 16
K = 32

BB = 1024
H_CHUNK = 64
CH = H_CHUNK * DIST
NEG = -3.4e38
POS = 3.4e38


def _tc_body(x_ref, d_ref, w_ref, b_ref, a_ref, o_ref):
    prox = lax.dot_general(
        x_ref[...], w_ref[...], (((1,), (1,)), ((), ())),
        preferred_element_type=jnp.float32)
    prox = prox + b_ref[...]

    d = d_ref[...]

    def step(i, carry):
        mpos, mneg = carry
        a_chunk = a_ref[:, pl.ds(i * CH, CH)]
        v = lax.dot_general(
            d, a_chunk, (((1,), (0,)), ((), ())),
            preferred_element_type=jnp.float32)
        hi = v[:, 0:DIST]
        lo = v[:, 0:DIST]
        for j in range(1, H_CHUNK):
            s = v[:, j * DIST:(j + 1) * DIST]
            hi = jnp.maximum(hi, s)
            lo = jnp.minimum(lo, s)
        return jnp.maximum(mpos, hi), jnp.minimum(mneg, lo)

    mpos, mneg = lax.fori_loop(
        0, H // H_CHUNK, step,
        (jnp.full((BB, DIST), NEG, jnp.float32),
         jnp.full((BB, DIST), POS, jnp.float32)))

    v = jnp.where(mpos >= -mneg, mpos, mneg)
    mod = 1.0 / (1.0 + jnp.exp(-v))
    res = prox * mod

    def peel(j, cur):
        m = jnp.max(cur, axis=1, keepdims=True)
        return jnp.where(cur == m, NEG, cur)

    cur = lax.fori_loop(0, K - 1, peel, res)
    thr = jnp.max(cur, axis=1, keepdims=True)
    o_ref[...] = jnp.where(res >= thr, res, 0.0)


@jax.jit
def _run(x_bf, d_bf, W_bf, b2d, A2_bf):
    return pl.pallas_call(
        _tc_body,
        grid=(B // BB,),
        in_specs=[
            pl.BlockSpec((BB, PROX), lambda i: (i, 0)),
            pl.BlockSpec((BB, DEN), lambda i: (i, 0)),
            pl.BlockSpec((H, PROX), lambda i: (0, 0)),
            pl.BlockSpec((1, H), lambda i: (0, 0)),
            pl.BlockSpec((DEN, H * DIST), lambda i: (0, 0)),
        ],
        out_specs=pl.BlockSpec((BB, H), lambda i: (i, 0)),
        out_shape=jax.ShapeDtypeStruct((B, H), jnp.float32),
    )(x_bf, d_bf, W_bf, b2d, A2_bf)


def kernel(proximal_input, distal_input, W, b, distal):
    A2 = jnp.transpose(distal, (1, 0, 2)).reshape(DEN, H * DIST)
    return _run(proximal_input.astype(jnp.bfloat16),
                distal_input.astype(jnp.bfloat16),
                W.astype(jnp.bfloat16),
                b.reshape(1, H),
                A2.astype(jnp.bfloat16))
